# Initial kernel scaffold; baseline (speedup 1.0000x reference)
import jax
import jax.numpy as jnp
from jax.experimental import pallas as pl


def kernel(x, table):
    raise NotImplementedError("write your pallas kernel here")


# speedup vs baseline: 2.4120x; 2.4120x over previous
import functools
import math

import jax
import jax.numpy as jnp
from jax import lax
from jax.experimental import pallas as pl
from jax.experimental.pallas import tpu as pltpu
from jax.experimental.pallas import tpu_sc as plsc

D_MODEL = 128
_SCALE = math.sqrt(128.0)

_NC = 2
_NS = 16
_NW = _NC * _NS
_LANES = 16

_CHUNK = 128


def _sc_embed(x32, table):
    nw, nchunk, chunk = x32.shape
    _, d = table.shape
    b = nw * nchunk * chunk

    mesh = plsc.VectorSubcoreMesh(core_axis_name="c", subcore_axis_name="s")

    @functools.partial(
        pl.kernel,
        mesh=mesh,
        out_type=jax.ShapeDtypeStruct((b, d), jnp.float32),
        scratch_types=[
            pltpu.VMEM((nchunk, chunk), jnp.int32),
            pltpu.VMEM((chunk, d), jnp.float32),
            pltpu.SemaphoreType.DMA,
        ],
    )
    def run(x_hbm, table_hbm, out_hbm, idx_v, rows_v, sem):
        wid = lax.axis_index("s") * _NC + lax.axis_index("c")
        base = wid * (nchunk * chunk)
        pltpu.sync_copy(x_hbm.at[wid], idx_v)

        def step(j, carry):
            pltpu.async_copy(table_hbm.at[idx_v.at[j]], rows_v, sem).wait()

            def scale_row(i, c2):
                for g in range(d // _LANES):
                    sl = pl.ds(g * _LANES, _LANES)
                    rows_v[i, sl] = rows_v[i, sl] * _SCALE
                return c2

            lax.fori_loop(0, chunk, scale_row, 0)
            pltpu.sync_copy(rows_v, out_hbm.at[pl.ds(base + j * chunk, chunk)])
            return carry

        lax.fori_loop(0, nchunk, step, 0)

    return run(x32, table)


def kernel(x, table):
    bsz, hist = x.shape
    b = bsz * hist
    x32 = x.reshape(_NW, b // (_NW * _CHUNK), _CHUNK).astype(jnp.int32)
    out = _sc_embed(x32, table.astype(jnp.float32))
    return out.reshape(bsz, hist, D_MODEL)


# speedup vs baseline: 2.9523x; 1.2240x over previous
import functools
import math

import jax
import jax.numpy as jnp
from jax import lax
from jax.experimental import pallas as pl
from jax.experimental.pallas import tpu as pltpu
from jax.experimental.pallas import tpu_sc as plsc

D_MODEL = 128
_SCALE = math.sqrt(128.0)

_NC = 2
_NS = 16
_NW = _NC * _NS
_LANES = 16

_CHUNK = 128
_NBUF = 5
_LEAD = 2


def _sc_embed(x32, table):
    nw, nchunk, chunk = x32.shape
    _, d = table.shape
    b = nw * nchunk * chunk
    assert nchunk % _NBUF == 0

    mesh = plsc.VectorSubcoreMesh(core_axis_name="c", subcore_axis_name="s")

    @functools.partial(
        pl.kernel,
        mesh=mesh,
        out_type=jax.ShapeDtypeStruct((b, d), jnp.float32),
        scratch_types=[
            pltpu.VMEM((nchunk, chunk), jnp.int32),
            pltpu.VMEM((_NBUF, chunk, d), jnp.float32),
            pltpu.SemaphoreType.DMA((_NBUF,)),
            pltpu.SemaphoreType.DMA((_NBUF,)),
        ],
    )
    def run(x_hbm, table_hbm, out_hbm, idx_v, rows_v, sem_g, sem_o):
        wid = lax.axis_index("s") * _NC + lax.axis_index("c")
        base = wid * (nchunk * chunk)
        pltpu.sync_copy(x_hbm.at[wid], idx_v)

        def g_copy(j, buf):
            return pltpu.make_async_copy(
                table_hbm.at[idx_v.at[j]], rows_v.at[buf], sem_g.at[buf])

        def o_copy(j, buf):
            return pltpu.make_async_copy(
                rows_v.at[buf], out_hbm.at[pl.ds(base + j * chunk, chunk)],
                sem_o.at[buf])

        for buf in range(_LEAD):
            g_copy(buf, buf).start()

        def outer(t, carry):
            j0 = t * _NBUF
            for bb in range(_NBUF):
                j = j0 + bb
                bg = (bb + _LEAD) % _NBUF

                @pl.when(j + _LEAD < nchunk)
                def _():
                    @pl.when(j >= _NBUF - _LEAD)
                    def _():
                        o_copy(j - (_NBUF - _LEAD), bg).wait()

                    g_copy(j + _LEAD, bg).start()

                g_copy(j, bb).wait()

                def scale_row(i, c2, bb=bb):
                    for grp in range(d // _LANES):
                        sl = pl.ds(grp * _LANES, _LANES)
                        rows_v[bb, i, sl] = rows_v[bb, i, sl] * _SCALE
                    return c2

                lax.fori_loop(0, chunk, scale_row, 0)
                o_copy(j, bb).start()
            return carry

        lax.fori_loop(0, nchunk // _NBUF, outer, 0)

        for bb in range(_NBUF):
            o_copy(nchunk - _NBUF + bb, bb).wait()

    return run(x32, table)


def kernel(x, table):
    bsz, hist = x.shape
    b = bsz * hist
    x32 = x.reshape(_NW, b // (_NW * _CHUNK), _CHUNK).astype(jnp.int32)
    out = _sc_embed(x32, table.astype(jnp.float32))
    return out.reshape(bsz, hist, D_MODEL)
